# Initial kernel scaffold; baseline (speedup 1.0000x reference)
import functools

import jax
import jax.numpy as jnp
from jax import lax
from jax.experimental import pallas as pl
from jax.experimental.pallas import tpu as pltpu
from jax.experimental.pallas import tpu_sc as plsc

N = 10000
E = 320000
D = 128
DH = 64
NC = 2
NS = 16
CH = 128
NCHUNK = E // CH
PER_SUB = NCHUNK // NS
REM = NCHUNK - PER_SUB * NS
RPS = N // NS

_sc_mesh = plsc.VectorSubcoreMesh(core_axis_name="c", subcore_axis_name="s")


def _sc_agg_body(h_ref, src_ref, dst_ref, z_ref, table, acc, src_v, dst_v, rows_v):
    c = lax.axis_index("c")
    s = lax.axis_index("s")
    r0 = s * RPS
    pltpu.sync_copy(h_ref.at[c, pl.ds(r0, RPS)], table.at[pl.ds(r0, RPS)])
    pltpu.sync_copy(h_ref.at[c, pl.ds(r0, RPS)], acc.at[pl.ds(r0, RPS)])
    plsc.subcore_barrier()

    def do_chunk(cid):
        base = cid * CH
        pltpu.sync_copy(src_ref.at[pl.ds(base, CH)], src_v)
        pltpu.sync_copy(dst_ref.at[pl.ds(base, CH)], dst_v)
        pltpu.sync_copy(table.at[src_v], rows_v)
        pltpu.sync_copy(rows_v, acc.at[dst_v], add=True)

    @pl.loop(0, PER_SUB)
    def _(i):
        do_chunk(i * NS + s)

    @pl.when(s < REM)
    def _():
        do_chunk(PER_SUB * NS + s)

    plsc.subcore_barrier()
    pltpu.sync_copy(acc.at[pl.ds(r0, RPS)], z_ref.at[c, pl.ds(r0, RPS)])


_sc_agg = pl.kernel(
    _sc_agg_body,
    out_type=jax.ShapeDtypeStruct((NC, N, DH), jnp.float32),
    mesh=_sc_mesh,
    scratch_types=[
        pltpu.VMEM_SHARED((N, DH), jnp.float32),
        pltpu.VMEM_SHARED((N, DH), jnp.float32),
        pltpu.VMEM((CH,), jnp.int32),
        pltpu.VMEM((CH,), jnp.int32),
        pltpu.VMEM((CH, DH), jnp.float32),
    ],
)


def _dense_body(split_out, relu_out, z_ref, w1_ref, b1_ref, g1_ref, be1_ref,
                w2_ref, b2_ref, go_ref, bo_ref, out_ref):
    z = jnp.concatenate([z_ref[0], z_ref[1]], axis=1)
    t = jnp.dot(z, w1_ref[...], preferred_element_type=jnp.float32) + b1_ref[...]
    mean = jnp.mean(t, axis=0)
    var = jnp.mean((t - mean) ** 2, axis=0)
    t = (t - mean) / jnp.sqrt(var + 1e-5) * g1_ref[...] + be1_ref[...]
    t = jnp.maximum(t, 0.0)
    u = jnp.dot(t, w2_ref[...], preferred_element_type=jnp.float32) + b2_ref[...]
    mean2 = jnp.mean(u, axis=0)
    var2 = jnp.mean((u - mean2) ** 2, axis=0)
    u = (u - mean2) / jnp.sqrt(var2 + 1e-5) * go_ref[...] + bo_ref[...]
    if relu_out:
        u = jnp.maximum(u, 0.0)
    if split_out:
        out_ref[0] = u[:, :DH]
        out_ref[1] = u[:, DH:]
    else:
        out_ref[...] = u


def _make_dense(split_out, relu_out):
    out_shape = (NC, N, DH) if split_out else (N, D)
    return pl.pallas_call(
        functools.partial(_dense_body, split_out, relu_out),
        out_shape=jax.ShapeDtypeStruct(out_shape, jnp.float32),
    )


_dense_mid = _make_dense(split_out=True, relu_out=True)
_dense_last = _make_dense(split_out=False, relu_out=False)


def kernel(x, edge_index, edge_attr, batch,
           W1_0, b1_0, g1_0, be1_0, W2_0, b2_0, go_0, bo_0,
           W1_1, b1_1, g1_1, be1_1, W2_1, b2_1, go_1, bo_1):
    src = edge_index[0]
    dst = edge_index[1]
    h2 = x.reshape(N, NC, DH).transpose(1, 0, 2)
    z2 = _sc_agg(h2, src, dst)
    h2 = _dense_mid(z2, W1_0, b1_0, g1_0, be1_0, W2_0, b2_0, go_0, bo_0)
    z2 = _sc_agg(h2, src, dst)
    return _dense_last(z2, W1_1, b1_1, g1_1, be1_1, W2_1, b2_1, go_1, bo_1)


# speedup vs baseline: 5.7159x; 5.7159x over previous
import functools

import jax
import jax.numpy as jnp
from jax import lax
from jax.experimental import pallas as pl
from jax.experimental.pallas import tpu as pltpu
from jax.experimental.pallas import tpu_sc as plsc

N = 10000
E = 320000
D = 128
NC = 2
NS = 16
CH = 128
EPC = E // NC
NCHUNK = EPC // CH
PER_SUB = NCHUNK // NS
REM = NCHUNK - PER_SUB * NS
RPS = 624
TAIL = N - NS * RPS


def _sc_agg_body(h_ref, src_ref, dst_ref, z_ref, acc, src_v, dst_v, rows_v):
    c = lax.axis_index("c")
    s = lax.axis_index("s")
    r0 = s * RPS
    pltpu.sync_copy(h_ref.at[pl.ds(r0, RPS)], acc.at[pl.ds(r0, RPS)])

    @pl.when(s == 0)
    def _():
        pltpu.sync_copy(h_ref.at[pl.ds(NS * RPS, TAIL)], acc.at[pl.ds(NS * RPS, TAIL)])

    plsc.subcore_barrier()

    def do_chunk(cid):
        base = cid * CH
        pltpu.sync_copy(src_ref.at[pl.ds(base, CH)], src_v)
        pltpu.sync_copy(dst_ref.at[pl.ds(base, CH)], dst_v)
        pltpu.sync_copy(h_ref.at[src_v], rows_v)
        pltpu.sync_copy(rows_v, acc.at[dst_v], add=True)

    @pl.loop(0, PER_SUB)
    def _(i):
        do_chunk(c * NCHUNK + i * NS + s)

    @pl.when(s < REM)
    def _():
        do_chunk(c * NCHUNK + PER_SUB * NS + s)

    plsc.subcore_barrier()
    pltpu.sync_copy(acc.at[pl.ds(r0, RPS)], z_ref.at[c, pl.ds(r0, RPS)])

    @pl.when(s == 0)
    def _():
        pltpu.sync_copy(acc.at[pl.ds(NS * RPS, TAIL)], z_ref.at[c, pl.ds(NS * RPS, TAIL)])


@functools.cache
def _get_sc_agg():
    mesh = plsc.VectorSubcoreMesh(
        core_axis_name="c", subcore_axis_name="s", num_cores=NC, num_subcores=NS)
    return pl.kernel(
        _sc_agg_body,
        out_type=jax.ShapeDtypeStruct((NC, N, D), jnp.float32),
        mesh=mesh,
        scratch_types=[
            pltpu.VMEM_SHARED((N, D), jnp.float32),
            pltpu.VMEM((CH,), jnp.int32),
            pltpu.VMEM((CH,), jnp.int32),
            pltpu.VMEM((CH, D), jnp.float32),
        ],
    )


def _dense_body(relu_out, z_ref, h_ref, w1_ref, b1_ref, g1_ref, be1_ref,
                w2_ref, b2_ref, go_ref, bo_ref, out_ref):
    z = z_ref[0] + z_ref[1] - h_ref[...]
    t = jnp.dot(z, w1_ref[...], preferred_element_type=jnp.float32) + b1_ref[...]
    mean = jnp.mean(t, axis=0)
    var = jnp.mean((t - mean) ** 2, axis=0)
    t = (t - mean) / jnp.sqrt(var + 1e-5) * g1_ref[...] + be1_ref[...]
    t = jnp.maximum(t, 0.0)
    u = jnp.dot(t, w2_ref[...], preferred_element_type=jnp.float32) + b2_ref[...]
    mean2 = jnp.mean(u, axis=0)
    var2 = jnp.mean((u - mean2) ** 2, axis=0)
    u = (u - mean2) / jnp.sqrt(var2 + 1e-5) * go_ref[...] + bo_ref[...]
    if relu_out:
        u = jnp.maximum(u, 0.0)
    out_ref[...] = u


def _make_dense(relu_out):
    return pl.pallas_call(
        functools.partial(_dense_body, relu_out),
        out_shape=jax.ShapeDtypeStruct((N, D), jnp.float32),
    )


_dense_mid = _make_dense(relu_out=True)
_dense_last = _make_dense(relu_out=False)


def kernel(x, edge_index, edge_attr, batch,
           W1_0, b1_0, g1_0, be1_0, W2_0, b2_0, go_0, bo_0,
           W1_1, b1_1, g1_1, be1_1, W2_1, b2_1, go_1, bo_1):
    src = edge_index[0]
    dst = edge_index[1]
    sc_agg = _get_sc_agg()
    p = sc_agg(x, src, dst)
    h = _dense_mid(p, x, W1_0, b1_0, g1_0, be1_0, W2_0, b2_0, go_0, bo_0)
    p = sc_agg(h, src, dst)
    return _dense_last(p, h, W1_1, b1_1, g1_1, be1_1, W2_1, b2_1, go_1, bo_1)


# speedup vs baseline: 8.7495x; 1.5307x over previous
import functools

import jax
import jax.numpy as jnp
from jax import lax
from jax.experimental import pallas as pl
from jax.experimental.pallas import tpu as pltpu
from jax.experimental.pallas import tpu_sc as plsc

N = 10000
E = 320000
D = 128
NC = 2
NS = 16
CH = 64
EPC = E // NC
NCHUNK = EPC // CH
PER_SUB = NCHUNK // NS
REM = NCHUNK - PER_SUB * NS
EPS = PER_SUB * CH
K = 4
NGRP = PER_SUB // K
RPS = 624
TAIL = N - NS * RPS


def _sc_agg_body(h_ref, src_ref, dst_ref, z_ref, acc, src_v, dst_v, rows_v,
                 esrc_v, edst_v, gsem, dsem, ssem):
    c = lax.axis_index("c")
    s = lax.axis_index("s")
    r0 = s * RPS
    e0 = c * EPC + s * EPS
    pltpu.sync_copy(h_ref.at[pl.ds(r0, RPS)], acc.at[pl.ds(r0, RPS)])
    pltpu.sync_copy(src_ref.at[pl.ds(e0, EPS)], src_v)

    @pl.when(s == 0)
    def _():
        pltpu.sync_copy(h_ref.at[pl.ds(NS * RPS, TAIL)], acc.at[pl.ds(NS * RPS, TAIL)])

    plsc.subcore_barrier()

    @pl.loop(0, NGRP)
    def _(i):
        i0 = i * K

        @pl.when(i > 0)
        def _():
            for k in range(K):
                pltpu.make_async_copy(
                    rows_v.at[k], acc.at[dst_v.at[k]], ssem).wait()

        for k in range(K):
            pltpu.async_copy(
                dst_ref.at[pl.ds(e0 + (i0 + k) * CH, CH)], dst_v.at[k], dsem)
            pltpu.async_copy(
                h_ref.at[src_v.at[pl.ds((i0 + k) * CH, CH)]], rows_v.at[k], gsem)
        for k in range(K):
            pltpu.make_async_copy(
                dst_ref.at[pl.ds(e0 + (i0 + k) * CH, CH)], dst_v.at[k], dsem).wait()
            pltpu.make_async_copy(
                h_ref.at[src_v.at[pl.ds((i0 + k) * CH, CH)]], rows_v.at[k], gsem).wait()
        for k in range(K):
            pltpu.async_copy(rows_v.at[k], acc.at[dst_v.at[k]], ssem, add=True)

    for k in range(K):
        pltpu.make_async_copy(rows_v.at[k], acc.at[dst_v.at[k]], ssem).wait()

    @pl.when(s < REM)
    def _():
        base = c * EPC + (NS * PER_SUB + s) * CH
        pltpu.sync_copy(src_ref.at[pl.ds(base, CH)], esrc_v)
        pltpu.sync_copy(dst_ref.at[pl.ds(base, CH)], edst_v)
        pltpu.sync_copy(h_ref.at[esrc_v], rows_v.at[0])
        pltpu.sync_copy(rows_v.at[0], acc.at[edst_v], add=True)

    plsc.subcore_barrier()
    pltpu.sync_copy(acc.at[pl.ds(r0, RPS)], z_ref.at[c, pl.ds(r0, RPS)])

    @pl.when(s == 0)
    def _():
        pltpu.sync_copy(acc.at[pl.ds(NS * RPS, TAIL)], z_ref.at[c, pl.ds(NS * RPS, TAIL)])


@functools.cache
def _get_sc_agg():
    mesh = plsc.VectorSubcoreMesh(
        core_axis_name="c", subcore_axis_name="s", num_cores=NC, num_subcores=NS)
    return pl.kernel(
        _sc_agg_body,
        out_type=jax.ShapeDtypeStruct((NC, N, D), jnp.float32),
        mesh=mesh,
        scratch_types=[
            pltpu.VMEM_SHARED((N, D), jnp.float32),
            pltpu.VMEM((EPS,), jnp.int32),
            pltpu.VMEM((K, CH), jnp.int32),
            pltpu.VMEM((K, CH, D), jnp.float32),
            pltpu.VMEM((CH,), jnp.int32),
            pltpu.VMEM((CH,), jnp.int32),
            pltpu.SemaphoreType.DMA,
            pltpu.SemaphoreType.DMA,
            pltpu.SemaphoreType.DMA,
        ],
    )


def _dense_body(relu_out, z_ref, h_ref, w1_ref, b1_ref, g1_ref, be1_ref,
                w2_ref, b2_ref, go_ref, bo_ref, out_ref):
    z = z_ref[0] + z_ref[1] - h_ref[...]
    t = jnp.dot(z, w1_ref[...], preferred_element_type=jnp.float32) + b1_ref[...]
    mean = jnp.mean(t, axis=0)
    var = jnp.mean((t - mean) ** 2, axis=0)
    t = (t - mean) / jnp.sqrt(var + 1e-5) * g1_ref[...] + be1_ref[...]
    t = jnp.maximum(t, 0.0)
    u = jnp.dot(t, w2_ref[...], preferred_element_type=jnp.float32) + b2_ref[...]
    mean2 = jnp.mean(u, axis=0)
    var2 = jnp.mean((u - mean2) ** 2, axis=0)
    u = (u - mean2) / jnp.sqrt(var2 + 1e-5) * go_ref[...] + bo_ref[...]
    if relu_out:
        u = jnp.maximum(u, 0.0)
    out_ref[...] = u


def _make_dense(relu_out):
    return pl.pallas_call(
        functools.partial(_dense_body, relu_out),
        out_shape=jax.ShapeDtypeStruct((N, D), jnp.float32),
    )


_dense_mid = _make_dense(relu_out=True)
_dense_last = _make_dense(relu_out=False)


def kernel(x, edge_index, edge_attr, batch,
           W1_0, b1_0, g1_0, be1_0, W2_0, b2_0, go_0, bo_0,
           W1_1, b1_1, g1_1, be1_1, W2_1, b2_1, go_1, bo_1):
    src = edge_index[0]
    dst = edge_index[1]
    sc_agg = _get_sc_agg()
    p = sc_agg(x, src, dst)
    h = _dense_mid(p, x, W1_0, b1_0, g1_0, be1_0, W2_0, b2_0, go_0, bo_0)
    p = sc_agg(h, src, dst)
    return _dense_last(p, h, W1_1, b1_1, g1_1, be1_1, W2_1, b2_1, go_1, bo_1)


# speedup vs baseline: 12.1389x; 1.3874x over previous
import functools

import jax
import jax.numpy as jnp
from jax import lax
from jax.experimental import pallas as pl
from jax.experimental.pallas import tpu as pltpu
from jax.experimental.pallas import tpu_sc as plsc

N = 10000
E = 320000
D = 128
NC = 2
NS = 16
CH = 96
EPC = E // NC
PER_SUB = 104
EPS = PER_SUB * CH
HALF = PER_SUB // 2
HCH = HALF * CH
NB = 3
REM_E = EPC - NS * EPS
REM_CH = 64
REM_W = REM_E // REM_CH
RPS = 624
TAIL = N - NS * RPS


def _sc_agg_body(h_ref, src_ref, dst_ref, z_ref, acc, src_v, dst_v, rows_v,
                 gsem, ssem):
    c = lax.axis_index("c")
    s = lax.axis_index("s")
    r0 = s * RPS
    e0 = c * EPC + s * EPS

    def issue_gather(g, b):
        pltpu.async_copy(
            h_ref.at[src_v.at[pl.ds(g * CH, CH)]], rows_v.at[b], gsem.at[b])

    def drain_gather(g, b):
        pltpu.make_async_copy(
            h_ref.at[src_v.at[pl.ds(g * CH, CH)]], rows_v.at[b], gsem.at[b]).wait()

    def issue_scatter(g, b):
        pltpu.async_copy(
            rows_v.at[b], acc.at[dst_v.at[pl.ds(g * CH, CH)]], ssem.at[b], add=True)

    def drain_scatter(g, b):
        pltpu.make_async_copy(
            rows_v.at[b], acc.at[dst_v.at[pl.ds(g * CH, CH)]], ssem.at[b]).wait()

    pltpu.sync_copy(h_ref.at[pl.ds(r0, RPS)], acc.at[pl.ds(r0, RPS)])

    @pl.when(s == 0)
    def _():
        pltpu.sync_copy(h_ref.at[pl.ds(NS * RPS, TAIL)], acc.at[pl.ds(NS * RPS, TAIL)])

    plsc.subcore_barrier()

    def run_half(h):
        base = e0 + h * HCH
        pltpu.sync_copy(src_ref.at[pl.ds(base, HCH)], src_v)
        pltpu.sync_copy(dst_ref.at[pl.ds(base, HCH)], dst_v)
        issue_gather(0, 0)
        issue_gather(1, 1)

        @pl.loop(0, (HALF - 4) // NB)
        def _(i):
            for j in range(NB):
                g = 2 + 3 * i + j
                b = (2 + j) % NB
                if j == 0:
                    @pl.when(i > 0)
                    def _():
                        drain_scatter(g - 3, b)
                else:
                    drain_scatter(g - 3, b)
                issue_gather(g, b)
                drain_gather(g - 2, j)
                issue_scatter(g - 2, j)

        for g in (50, 51):
            drain_scatter(g - 3, g % NB)
            issue_gather(g, g % NB)
            drain_gather(g - 2, (g - 2) % NB)
            issue_scatter(g - 2, (g - 2) % NB)
        for g in (50, 51):
            drain_gather(g, g % NB)
            issue_scatter(g, g % NB)
        for g in (49, 50, 51):
            drain_scatter(g, g % NB)

    run_half(0)
    run_half(1)

    @pl.when(s < REM_W)
    def _():
        base = c * EPC + NS * EPS + s * REM_CH
        pltpu.sync_copy(src_ref.at[pl.ds(base, REM_CH)], src_v.at[pl.ds(0, REM_CH)])
        pltpu.sync_copy(dst_ref.at[pl.ds(base, REM_CH)], dst_v.at[pl.ds(0, REM_CH)])
        pltpu.sync_copy(h_ref.at[src_v.at[pl.ds(0, REM_CH)]],
                        rows_v.at[0, pl.ds(0, REM_CH)])
        pltpu.sync_copy(rows_v.at[0, pl.ds(0, REM_CH)],
                        acc.at[dst_v.at[pl.ds(0, REM_CH)]], add=True)

    plsc.subcore_barrier()
    pltpu.sync_copy(acc.at[pl.ds(r0, RPS)], z_ref.at[c, pl.ds(r0, RPS)])

    @pl.when(s == 0)
    def _():
        pltpu.sync_copy(acc.at[pl.ds(NS * RPS, TAIL)], z_ref.at[c, pl.ds(NS * RPS, TAIL)])


@functools.cache
def _get_sc_agg():
    mesh = plsc.VectorSubcoreMesh(
        core_axis_name="c", subcore_axis_name="s", num_cores=NC, num_subcores=NS)
    return pl.kernel(
        _sc_agg_body,
        out_type=jax.ShapeDtypeStruct((NC, N, D), jnp.float32),
        mesh=mesh,
        scratch_types=[
            pltpu.VMEM_SHARED((N, D), jnp.float32),
            pltpu.VMEM((HCH,), jnp.int32),
            pltpu.VMEM((HCH,), jnp.int32),
            pltpu.VMEM((NB, CH, D), jnp.float32),
            pltpu.SemaphoreType.DMA((NB,)),
            pltpu.SemaphoreType.DMA((NB,)),
        ],
    )


def _dense_body(relu_out, z_ref, h_ref, w1_ref, b1_ref, g1_ref, be1_ref,
                w2_ref, b2_ref, go_ref, bo_ref, out_ref):
    z = z_ref[0] + z_ref[1] - h_ref[...]
    t = jnp.dot(z, w1_ref[...], preferred_element_type=jnp.float32) + b1_ref[...]
    mean = jnp.mean(t, axis=0)
    var = jnp.mean((t - mean) ** 2, axis=0)
    t = (t - mean) / jnp.sqrt(var + 1e-5) * g1_ref[...] + be1_ref[...]
    t = jnp.maximum(t, 0.0)
    u = jnp.dot(t, w2_ref[...], preferred_element_type=jnp.float32) + b2_ref[...]
    mean2 = jnp.mean(u, axis=0)
    var2 = jnp.mean((u - mean2) ** 2, axis=0)
    u = (u - mean2) / jnp.sqrt(var2 + 1e-5) * go_ref[...] + bo_ref[...]
    if relu_out:
        u = jnp.maximum(u, 0.0)
    out_ref[...] = u


def _make_dense(relu_out):
    return pl.pallas_call(
        functools.partial(_dense_body, relu_out),
        out_shape=jax.ShapeDtypeStruct((N, D), jnp.float32),
    )


_dense_mid = _make_dense(relu_out=True)
_dense_last = _make_dense(relu_out=False)


def kernel(x, edge_index, edge_attr, batch,
           W1_0, b1_0, g1_0, be1_0, W2_0, b2_0, go_0, bo_0,
           W1_1, b1_1, g1_1, be1_1, W2_1, b2_1, go_1, bo_1):
    src = edge_index[0]
    dst = edge_index[1]
    sc_agg = _get_sc_agg()
    p = sc_agg(x, src, dst)
    h = _dense_mid(p, x, W1_0, b1_0, g1_0, be1_0, W2_0, b2_0, go_0, bo_0)
    p = sc_agg(h, src, dst)
    return _dense_last(p, h, W1_1, b1_1, g1_1, be1_1, W2_1, b2_1, go_1, bo_1)


# speedup vs baseline: 12.7195x; 1.0478x over previous
import functools

import jax
import jax.numpy as jnp
from jax import lax
from jax.experimental import pallas as pl
from jax.experimental.pallas import tpu as pltpu
from jax.experimental.pallas import tpu_sc as plsc

N = 10000
E = 320000
D = 128
NC = 2
NS = 16
CH = 96
EPC = E // NC
PER_SUB = 104
EPS = PER_SUB * CH
HALF = PER_SUB // 2
HCH = HALF * CH
NB = 3
REM_E = EPC - NS * EPS
REM_CH = 64
REM_W = REM_E // REM_CH
RPS = 624
TAIL = N - NS * RPS


def _sc_agg_body(h_ref, e_ref, z_ref, acc, src_v, dst_v, rows_v,
                 gsem, ssem):
    src_ref = e_ref.at[0]
    dst_ref = e_ref.at[1]
    c = lax.axis_index("c")
    s = lax.axis_index("s")
    r0 = s * RPS
    e0 = c * EPC + s * EPS

    def issue_gather(g, b):
        pltpu.async_copy(
            h_ref.at[src_v.at[pl.ds(g * CH, CH)]], rows_v.at[b], gsem.at[b])

    def drain_gather(g, b):
        pltpu.make_async_copy(
            h_ref.at[src_v.at[pl.ds(g * CH, CH)]], rows_v.at[b], gsem.at[b]).wait()

    def issue_scatter(g, b):
        pltpu.async_copy(
            rows_v.at[b], acc.at[dst_v.at[pl.ds(g * CH, CH)]], ssem.at[b], add=True)

    def drain_scatter(g, b):
        pltpu.make_async_copy(
            rows_v.at[b], acc.at[dst_v.at[pl.ds(g * CH, CH)]], ssem.at[b]).wait()

    pltpu.sync_copy(h_ref.at[pl.ds(r0, RPS)], acc.at[pl.ds(r0, RPS)])

    @pl.when(s == 0)
    def _():
        pltpu.sync_copy(h_ref.at[pl.ds(NS * RPS, TAIL)], acc.at[pl.ds(NS * RPS, TAIL)])

    plsc.subcore_barrier()

    def run_half(h):
        base = e0 + h * HCH
        pltpu.sync_copy(src_ref.at[pl.ds(base, HCH)], src_v)
        pltpu.sync_copy(dst_ref.at[pl.ds(base, HCH)], dst_v)
        issue_gather(0, 0)
        issue_gather(1, 1)

        @pl.loop(0, (HALF - 4) // NB)
        def _(i):
            for j in range(NB):
                g = 2 + 3 * i + j
                b = (2 + j) % NB
                if j == 0:
                    @pl.when(i > 0)
                    def _():
                        drain_scatter(g - 3, b)
                else:
                    drain_scatter(g - 3, b)
                issue_gather(g, b)
                drain_gather(g - 2, j)
                issue_scatter(g - 2, j)

        for g in (50, 51):
            drain_scatter(g - 3, g % NB)
            issue_gather(g, g % NB)
            drain_gather(g - 2, (g - 2) % NB)
            issue_scatter(g - 2, (g - 2) % NB)
        for g in (50, 51):
            drain_gather(g, g % NB)
            issue_scatter(g, g % NB)
        for g in (49, 50, 51):
            drain_scatter(g, g % NB)

    run_half(0)
    run_half(1)

    @pl.when(s < REM_W)
    def _():
        base = c * EPC + NS * EPS + s * REM_CH
        pltpu.sync_copy(src_ref.at[pl.ds(base, REM_CH)], src_v.at[pl.ds(0, REM_CH)])
        pltpu.sync_copy(dst_ref.at[pl.ds(base, REM_CH)], dst_v.at[pl.ds(0, REM_CH)])
        pltpu.sync_copy(h_ref.at[src_v.at[pl.ds(0, REM_CH)]],
                        rows_v.at[0, pl.ds(0, REM_CH)])
        pltpu.sync_copy(rows_v.at[0, pl.ds(0, REM_CH)],
                        acc.at[dst_v.at[pl.ds(0, REM_CH)]], add=True)

    plsc.subcore_barrier()
    pltpu.sync_copy(acc.at[pl.ds(r0, RPS)], z_ref.at[c, pl.ds(r0, RPS)])

    @pl.when(s == 0)
    def _():
        pltpu.sync_copy(acc.at[pl.ds(NS * RPS, TAIL)], z_ref.at[c, pl.ds(NS * RPS, TAIL)])


@functools.cache
def _get_sc_agg():
    mesh = plsc.VectorSubcoreMesh(
        core_axis_name="c", subcore_axis_name="s", num_cores=NC, num_subcores=NS)
    return pl.kernel(
        _sc_agg_body,
        out_type=jax.ShapeDtypeStruct((NC, N, D), jnp.float32),
        mesh=mesh,
        scratch_types=[
            pltpu.VMEM_SHARED((N, D), jnp.float32),
            pltpu.VMEM((HCH,), jnp.int32),
            pltpu.VMEM((HCH,), jnp.int32),
            pltpu.VMEM((NB, CH, D), jnp.float32),
            pltpu.SemaphoreType.DMA((NB,)),
            pltpu.SemaphoreType.DMA((NB,)),
        ],
    )


def _dense_body(relu_out, z_ref, h_ref, w1_ref, b1_ref, g1_ref, be1_ref,
                w2_ref, b2_ref, go_ref, bo_ref, out_ref):
    z = z_ref[0] + z_ref[1] - h_ref[...]
    t = jnp.dot(z, w1_ref[...], preferred_element_type=jnp.float32) + b1_ref[...]
    mean = jnp.mean(t, axis=0)
    var = jnp.mean((t - mean) ** 2, axis=0)
    t = (t - mean) / jnp.sqrt(var + 1e-5) * g1_ref[...] + be1_ref[...]
    t = jnp.maximum(t, 0.0)
    u = jnp.dot(t, w2_ref[...], preferred_element_type=jnp.float32) + b2_ref[...]
    mean2 = jnp.mean(u, axis=0)
    var2 = jnp.mean((u - mean2) ** 2, axis=0)
    u = (u - mean2) / jnp.sqrt(var2 + 1e-5) * go_ref[...] + bo_ref[...]
    if relu_out:
        u = jnp.maximum(u, 0.0)
    out_ref[...] = u


def _make_dense(relu_out):
    return pl.pallas_call(
        functools.partial(_dense_body, relu_out),
        out_shape=jax.ShapeDtypeStruct((N, D), jnp.float32),
    )


_dense_mid = _make_dense(relu_out=True)
_dense_last = _make_dense(relu_out=False)


def kernel(x, edge_index, edge_attr, batch,
           W1_0, b1_0, g1_0, be1_0, W2_0, b2_0, go_0, bo_0,
           W1_1, b1_1, g1_1, be1_1, W2_1, b2_1, go_1, bo_1):
    sc_agg = _get_sc_agg()
    p = sc_agg(x, edge_index)
    h = _dense_mid(p, x, W1_0, b1_0, g1_0, be1_0, W2_0, b2_0, go_0, bo_0)
    p = sc_agg(h, edge_index)
    return _dense_last(p, h, W1_1, b1_1, g1_1, be1_1, W2_1, b2_1, go_1, bo_1)
